# speedup vs baseline: 13.4394x; 13.4394x over previous
import jax
import jax.numpy as jnp
from jax import lax
from jax.experimental import pallas as pl

B_BLK = 4096


def _body(x_ref, hard_ref, soft_ref):
    x = x_ref[0]
    m = jnp.max(x, axis=-1, keepdims=True)
    e = jnp.exp(x - m)
    s = jnp.sum(e, axis=-1, keepdims=True)
    p = e / s
    soft_ref[0] = p
    am = jnp.argmax(p, axis=-1).astype(jnp.int32)
    k_iota = lax.broadcasted_iota(jnp.int32, (16, B_BLK), 0)
    hard_ref[0] = jnp.where(k_iota == am[None, :], 1.0, 0.0)


def kernel(mask):
    bshape = mask.shape
    b, n_last = bshape[0], bshape[-1]
    n = bshape[1] * bshape[2] * bshape[3]
    x = mask.reshape(b, n, n_last)

    hard, soft = pl.pallas_call(
        _body,
        grid=(b, n // B_BLK),
        in_specs=[pl.BlockSpec((1, B_BLK, n_last), lambda i, j: (i, j, 0))],
        out_specs=[
            pl.BlockSpec((1, n_last, B_BLK), lambda i, j: (i, 0, j)),
            pl.BlockSpec((1, B_BLK, n_last), lambda i, j: (i, j, 0)),
        ],
        out_shape=[
            jax.ShapeDtypeStruct((b, n_last, n), jnp.float32),
            jax.ShapeDtypeStruct((b, n, n_last), jnp.float32),
        ],
    )(x)

    hard_mask = hard.reshape(b, n_last, bshape[1], bshape[2], bshape[3])
    y_soft = soft.reshape(b * n, n_last)
    return (hard_mask, y_soft)
